# baseline (speedup 1.0000x reference)
import functools
import math

import jax
import jax.numpy as jnp
from jax import lax
from jax.experimental import pallas as pl
from jax.experimental.pallas import tpu as pltpu
from jax.experimental.pallas import tpu_sc as plsc

_DK = 16
_H = 8
_ACCW = 144
_EB = 32
_NW = 32


def _qkv_pallas(x, wq, bq, qscale, wk, bk, ratt_bd, wv, bv, rmsg_bd, *, interpret=False):
    n, d = x.shape
    blk = 1000
    hi = lax.Precision.HIGHEST
    dn = (((1,), (1,)), ((), ()))

    def body(x_ref, wq_ref, bq_ref, qs_ref, wk_ref, bk_ref, ra_ref, wv_ref,
             bv_ref, rm_ref, q_out, k_out, v_out):
        xb = x_ref[...]
        q = lax.dot_general(xb, wq_ref[...], dn, precision=hi)
        q_out[...] = (q + bq_ref[...]) * qs_ref[...]
        k = lax.dot_general(xb, wk_ref[...], dn, precision=hi) + bk_ref[...]
        k_out[...] = jnp.dot(k, ra_ref[...], precision=hi)
        v = lax.dot_general(xb, wv_ref[...], dn, precision=hi) + bv_ref[...]
        v_out[...] = jnp.dot(v, rm_ref[...], precision=hi)

    def full(shape):
        return pl.BlockSpec(shape, lambda i: tuple(0 for _ in shape))

    return pl.pallas_call(
        body,
        grid=(n // blk,),
        in_specs=[
            pl.BlockSpec((blk, d), lambda i: (i, 0)),
            full((d, d)), full((1, d)), full((1, d)),
            full((d, d)), full((1, d)), full((d, d)),
            full((d, d)), full((1, d)), full((d, d)),
        ],
        out_specs=[
            pl.BlockSpec((blk, d), lambda i: (i, 0)),
            pl.BlockSpec((blk, d), lambda i: (i, 0)),
            pl.BlockSpec((blk, d), lambda i: (i, 0)),
        ],
        out_shape=[
            jax.ShapeDtypeStruct((n, d), jnp.float32),
            jax.ShapeDtypeStruct((n, d), jnp.float32),
            jax.ShapeDtypeStruct((n, d), jnp.float32),
        ],
        interpret=interpret,
    )(x, wq, bq.reshape(1, d), qscale.reshape(1, d), wk, bk.reshape(1, d),
      ratt_bd, wv, bv.reshape(1, d), rmsg_bd)


def _final_pallas(num, den, x, wa, ba, skip, *, interpret=False):
    n, d = x.shape
    blk = 1000
    hi = lax.Precision.HIGHEST
    dn = (((1,), (1,)), ((), ()))

    def body(num_ref, den_ref, x_ref, wa_ref, ba_ref, skip_ref, out_ref):
        nm = num_ref[0] + num_ref[1]
        den8 = den_ref[0] + den_ref[1]
        hh = lax.broadcasted_iota(jnp.int32, (_H, d), 0)
        cc = lax.broadcasted_iota(jnp.int32, (_H, d), 1)
        sel = jnp.where((cc // _DK) == hh, 1.0, 0.0)
        den_rep = jnp.dot(den8, sel, precision=hi)
        den_rep = jnp.where(den_rep > 0.0, den_rep, 1.0)
        t = nm / den_rep
        out = lax.dot_general(t, wa_ref[...], dn, precision=hi) + ba_ref[...]
        alpha = 1.0 / (1.0 + jnp.exp(-skip_ref[...]))
        out_ref[...] = out * alpha + x_ref[...] * (1.0 - alpha)

    return pl.pallas_call(
        body,
        grid=(n // blk,),
        in_specs=[
            pl.BlockSpec((2, blk, d), lambda i: (0, i, 0)),
            pl.BlockSpec((2, blk, _H), lambda i: (0, i, 0)),
            pl.BlockSpec((blk, d), lambda i: (i, 0)),
            pl.BlockSpec((d, d), lambda i: (0, 0)),
            pl.BlockSpec((1, d), lambda i: (0, 0)),
            pl.BlockSpec((1, 1), lambda i: (0, 0)),
        ],
        out_specs=pl.BlockSpec((blk, d), lambda i: (i, 0)),
        out_shape=jax.ShapeDtypeStruct((n, d), jnp.float32),
        interpret=interpret,
    )(num, den, x, wa, ba.reshape(1, d), skip.reshape(1, 1))


def _edge_call(pairs, q_eff, k_eff, v_eff, n, e_real):
    e_pad = pairs.shape[0] - 2 * _EB
    d = q_eff.shape[1]
    nblk_w = e_pad // (_EB * _NW)
    assert nblk_w * _EB * _NW == e_pad and nblk_w % 2 == 1 and nblk_w >= 3
    assert e_real % _EB == 0
    npairs = (nblk_w - 1) // 2
    nden = (n + 15) // 16
    r_acc = (n + nden + 7) // 8 * 8
    rows_per = (r_acc // 16) // 8 * 8
    tail = r_acc - rows_per * 16
    assert rows_per % 8 == 0 and tail % 8 == 0
    mesh = plsc.VectorSubcoreMesh(core_axis_name="c", subcore_axis_name="s")

    ng = _EB // 16

    @functools.partial(
        pl.kernel,
        out_type=jax.ShapeDtypeStruct((2, r_acc, 128), jnp.float32),
        mesh=mesh,
        scratch_types=[
            pltpu.VMEM((_EB, 2), jnp.int32),
            [pltpu.VMEM((_EB,), jnp.int32)] * 2,
            [pltpu.VMEM((_EB,), jnp.int32)] * 2,
            [pltpu.VMEM((_EB, 128), jnp.float32)] * 2,
            [pltpu.VMEM((_EB, 128), jnp.float32)] * 2,
            pltpu.VMEM((_EB, 128), jnp.float32),
            [pltpu.VMEM((2 * _EB, 128), jnp.float32)] * 2,
            [pltpu.VMEM((2 * _EB,), jnp.int32)] * 2,
            pltpu.VMEM_SHARED((r_acc, 128), jnp.float32),
            pltpu.SemaphoreType.DMA,
            [pltpu.SemaphoreType.DMA] * 2,
            [pltpu.SemaphoreType.DMA] * 2,
            pltpu.SemaphoreType.DMA,
            [pltpu.SemaphoreType.DMA] * 2,
        ],
        compiler_params=pltpu.CompilerParams(needs_layout_passes=False),
    )
    def edge_kernel(ep_hbm, q_hbm, k_hbm, v_hbm, out_hbm,
                    ep, giq, gis, qv, kv, vv, sdv, sdi, acc,
                    isem, gqs, gks, gvs, sss):
        cid = lax.axis_index("c")
        sid = lax.axis_index("s")
        wid = sid * 2 + cid
        zeros16 = jnp.zeros((16,), jnp.float32)
        lanes = jnp.arange(16, dtype=jnp.int32)

        def zrow(r, _):
            for j in range(8):
                sdv[0][r, pl.ds(j * 16, 16)] = zeros16
                sdv[1][r, pl.ds(j * 16, 16)] = zeros16
            return 0
        lax.fori_loop(0, 2 * _EB, zrow, 0)

        row0 = pl.multiple_of(sid * rows_per, 8)
        off = 0
        while off < rows_per:
            c = min(2 * _EB, rows_per - off)
            pltpu.sync_copy(sdv[0].at[pl.ds(0, c)], acc.at[pl.ds(row0 + off, c)])
            off += c
        if tail:
            @pl.when(sid == 0)
            def _zt():
                pltpu.sync_copy(sdv[0].at[pl.ds(0, tail)],
                                acc.at[pl.ds(rows_per * 16, tail)])
        plsc.subcore_barrier()

        def idx_fetch(t):
            ebase = pl.multiple_of((wid * nblk_w + t) * _EB, 8)
            pltpu.async_copy(ep_hbm.at[pl.ds(ebase, _EB)], ep, isem)

        def slot(t, p):
            pltpu.make_async_copy(ep_hbm.at[pl.ds(0, _EB)], ep, isem).wait()
            c0 = jnp.full((16,), 0, jnp.int32)
            c1 = jnp.full((16,), 1, jnp.int32)
            for g in range(ng):
                rows = g * 16 + lanes
                gis[p][pl.ds(g * 16, 16)] = plsc.load_gather(ep, [rows, c0])
                giq[p][pl.ds(g * 16, 16)] = plsc.load_gather(ep, [rows, c1])
            pltpu.async_copy(q_hbm.at[giq[p]], qv[p], gqs[p])
            pltpu.async_copy(k_hbm.at[gis[p]], kv[p], gks[p])
            idx_fetch(t + 1)

        def wait_scatter(p):
            pltpu.make_async_copy(sdv[p], acc.at[sdi[p]], sss[p]).wait()

        def _treesum(vals):
            while len(vals) > 1:
                vals = [a + b for a, b in zip(vals[::2], vals[1::2])]
            return vals[0]

        def process(t, p, first):
            if not first:
                wait_scatter(p)
                for g in range(ng):
                    dold = sdi[p][pl.ds(g * 16, 16)]
                    dcol = (dold & 15) * 8
                    rows = _EB + g * 16 + lanes
                    for h in range(_H):
                        plsc.store_scatter(sdv[p], [rows, dcol + h], zeros16)
            pltpu.make_async_copy(q_hbm.at[giq[p]], qv[p], gqs[p]).wait()
            pltpu.make_async_copy(k_hbm.at[gis[p]], kv[p], gks[p]).wait()
            pltpu.make_async_copy(v_hbm.at[gis[p]], vv, gvs).wait()
            padf = jnp.where((wid * nblk_w + t) * _EB < e_real, 1.0, 0.0)
            for g in range(ng):
                rs = pl.ds(g * 16, 16)
                rows = g * 16 + lanes
                dlan = giq[p][rs]
                sdi[p][rs] = dlan
                sdi[p][pl.ds(_EB + g * 16, 16)] = n + lax.shift_right_logical(dlan, 4)
                dcol = (dlan & 15) * 8

                def _edge(hh, _):
                    for u in range(2):
                        h = hh * 2 + u
                        c0 = h * _DK
                        parts = []
                        for c in range(4):
                            cs = [jnp.full((16,), 4 * c + j, jnp.int32) + c0
                                  for j in range(4)]
                            qs = [plsc.load_gather(qv[p], [rows, cj]) for cj in cs]
                            ks = [plsc.load_gather(kv[p], [rows, cj]) for cj in cs]
                            parts.append(_treesum([a * b for a, b in zip(qs, ks)]))
                        ex = jnp.exp(_treesum(parts)) * padf
                        plsc.store_scatter(sdv[p], [_EB + rows, dcol + h], ex)
                        for c in range(4):
                            cs = [jnp.full((16,), 4 * c + j, jnp.int32) + c0
                                  for j in range(4)]
                            vs = [plsc.load_gather(vv, [rows, cj]) for cj in cs]
                            for j in range(4):
                                plsc.store_scatter(sdv[p], [rows, cs[j]], vs[j] * ex)
                    return 0
                lax.fori_loop(0, _H // 2, _edge, 0)
            pltpu.async_copy(sdv[p], acc.at[sdi[p]], sss[p], add=True)
            pltpu.async_copy(v_hbm.at[gis[1 - p]], vv, gvs)

        idx_fetch(0)
        slot(0, 0)
        pltpu.async_copy(v_hbm.at[gis[0]], vv, gvs)
        slot(1, 1)
        process(0, 0, True)
        slot(2, 0)
        process(1, 1, True)

        def pair(i, _):
            slot(2 * i + 1, 1)
            process(2 * i, 0, False)
            slot(2 * i + 2, 0)
            process(2 * i + 1, 1, False)
            return 0
        lax.fori_loop(1, npairs, pair, 0)
        process(nblk_w - 1, 0, False)
        wait_scatter(0)
        wait_scatter(1)
        pltpu.make_async_copy(v_hbm.at[gis[1]], vv, gvs).wait()
        pltpu.make_async_copy(ep_hbm.at[pl.ds(0, _EB)], ep, isem).wait()

        plsc.subcore_barrier()
        pltpu.sync_copy(acc.at[pl.ds(row0, rows_per)],
                        out_hbm.at[cid, pl.ds(row0, rows_per)])
        if tail:
            @pl.when(sid == 0)
            def _ct():
                pltpu.sync_copy(acc.at[pl.ds(rows_per * 16, tail)],
                                out_hbm.at[cid, pl.ds(rows_per * 16, tail)])

    return edge_kernel(pairs, q_eff, k_eff, v_eff)


def kernel(x, edge_index, Wk, bk, Wq, bq, Wv, bv, Wa, ba, rel_att, rel_pri, rel_msg, skip):
    n, d = x.shape
    h, dk, _ = rel_att.shape
    ratt_bd = jax.scipy.linalg.block_diag(*[rel_att[i] for i in range(h)])
    rmsg_bd = jax.scipy.linalg.block_diag(*[rel_msg[i] for i in range(h)])
    qscale = jnp.repeat(rel_pri, dk) / math.sqrt(dk)
    q_eff, k_eff, v_eff = _qkv_pallas(x, Wq, bq, qscale, Wk, bk, ratt_bd, Wv, bv, rmsg_bd)
    e = edge_index.shape[1]
    grp_e = _EB * _NW
    nblk_w = -(-e // grp_e)
    if nblk_w % 2 == 0:
        nblk_w += 1
    e_pad = nblk_w * grp_e
    pairs = jnp.zeros((e_pad + 2 * _EB, 2), jnp.int32)
    pairs = pairs.at[:e, 0].set(edge_index[0]).at[:e, 1].set(edge_index[1])
    acc = _edge_call(pairs, q_eff, k_eff, v_eff, n, e)
    nden = (n + 15) // 16
    den = acc[:, n:n + nden, :].reshape(2, nden * 16, _H)[:, :n, :]
    return _final_pallas(acc[:, :n, :], den, x, Wa, ba, skip)


# speedup vs baseline: 1.0788x; 1.0788x over previous
import functools
import math

import jax
import jax.numpy as jnp
from jax import lax
from jax.experimental import pallas as pl
from jax.experimental.pallas import tpu as pltpu
from jax.experimental.pallas import tpu_sc as plsc

_DK = 16
_H = 8
_ACCW = 144
_EB = 32
_NW = 32


def _qkv_pallas(x, wq, bq, qscale, wk, bk, ratt_bd, wv, bv, rmsg_bd, *, interpret=False):
    n, d = x.shape
    blk = 1000
    hi = lax.Precision.HIGHEST
    dn = (((1,), (1,)), ((), ()))

    def body(x_ref, wq_ref, bq_ref, qs_ref, wk_ref, bk_ref, ra_ref, wv_ref,
             bv_ref, rm_ref, q_out, k_out, v_out):
        xb = x_ref[...]
        q = lax.dot_general(xb, wq_ref[...], dn, precision=hi)
        q_out[...] = (q + bq_ref[...]) * qs_ref[...]
        k = lax.dot_general(xb, wk_ref[...], dn, precision=hi) + bk_ref[...]
        k_out[...] = jnp.dot(k, ra_ref[...], precision=hi)
        v = lax.dot_general(xb, wv_ref[...], dn, precision=hi) + bv_ref[...]
        v_out[...] = jnp.dot(v, rm_ref[...], precision=hi)

    def full(shape):
        return pl.BlockSpec(shape, lambda i: tuple(0 for _ in shape))

    return pl.pallas_call(
        body,
        grid=(n // blk,),
        in_specs=[
            pl.BlockSpec((blk, d), lambda i: (i, 0)),
            full((d, d)), full((1, d)), full((1, d)),
            full((d, d)), full((1, d)), full((d, d)),
            full((d, d)), full((1, d)), full((d, d)),
        ],
        out_specs=[
            pl.BlockSpec((blk, d), lambda i: (i, 0)),
            pl.BlockSpec((blk, d), lambda i: (i, 0)),
            pl.BlockSpec((blk, d), lambda i: (i, 0)),
        ],
        out_shape=[
            jax.ShapeDtypeStruct((n, d), jnp.float32),
            jax.ShapeDtypeStruct((n, d), jnp.float32),
            jax.ShapeDtypeStruct((n, d), jnp.float32),
        ],
        interpret=interpret,
    )(x, wq, bq.reshape(1, d), qscale.reshape(1, d), wk, bk.reshape(1, d),
      ratt_bd, wv, bv.reshape(1, d), rmsg_bd)


def _final_pallas(num, den, x, wa, ba, skip, *, interpret=False):
    n, d = x.shape
    blk = 1000
    hi = lax.Precision.HIGHEST
    dn = (((1,), (1,)), ((), ()))

    def body(num_ref, den_ref, x_ref, wa_ref, ba_ref, skip_ref, out_ref):
        nm = num_ref[0] + num_ref[1]
        den8 = den_ref[0] + den_ref[1]
        hh = lax.broadcasted_iota(jnp.int32, (_H, d), 0)
        cc = lax.broadcasted_iota(jnp.int32, (_H, d), 1)
        sel = jnp.where((cc // _DK) == hh, 1.0, 0.0)
        den_rep = jnp.dot(den8, sel, precision=hi)
        den_rep = jnp.where(den_rep > 0.0, den_rep, 1.0)
        t = nm / den_rep
        out = lax.dot_general(t, wa_ref[...], dn, precision=hi) + ba_ref[...]
        alpha = 1.0 / (1.0 + jnp.exp(-skip_ref[...]))
        out_ref[...] = out * alpha + x_ref[...] * (1.0 - alpha)

    return pl.pallas_call(
        body,
        grid=(n // blk,),
        in_specs=[
            pl.BlockSpec((2, blk, d), lambda i: (0, i, 0)),
            pl.BlockSpec((2, blk, _H), lambda i: (0, i, 0)),
            pl.BlockSpec((blk, d), lambda i: (i, 0)),
            pl.BlockSpec((d, d), lambda i: (0, 0)),
            pl.BlockSpec((1, d), lambda i: (0, 0)),
            pl.BlockSpec((1, 1), lambda i: (0, 0)),
        ],
        out_specs=pl.BlockSpec((blk, d), lambda i: (i, 0)),
        out_shape=jax.ShapeDtypeStruct((n, d), jnp.float32),
        interpret=interpret,
    )(num, den, x, wa, ba.reshape(1, d), skip.reshape(1, 1))


def _edge_call(pairs, q_eff, k_eff, v_eff, n, e_real):
    e_pad = pairs.shape[0] - 2 * _EB
    d = q_eff.shape[1]
    nblk_w = e_pad // (_EB * _NW)
    assert nblk_w * _EB * _NW == e_pad and nblk_w % 2 == 1 and nblk_w >= 3
    assert e_real % _EB == 0
    npairs = (nblk_w - 1) // 2
    nden = (n + 15) // 16
    r_acc = (n + nden + 7) // 8 * 8
    rows_per = (r_acc // 16) // 8 * 8
    tail = r_acc - rows_per * 16
    assert rows_per % 8 == 0 and tail % 8 == 0
    mesh = plsc.VectorSubcoreMesh(core_axis_name="c", subcore_axis_name="s")

    ng = _EB // 16

    @functools.partial(
        pl.kernel,
        out_type=jax.ShapeDtypeStruct((2, r_acc, 128), jnp.float32),
        mesh=mesh,
        scratch_types=[
            pltpu.VMEM((_EB, 2), jnp.int32),
            [pltpu.VMEM((_EB,), jnp.int32)] * 2,
            [pltpu.VMEM((_EB,), jnp.int32)] * 2,
            [pltpu.VMEM((_EB, 128), jnp.float32)] * 2,
            [pltpu.VMEM((_EB, 128), jnp.float32)] * 2,
            pltpu.VMEM((_EB, 128), jnp.float32),
            [pltpu.VMEM((2 * _EB, 128), jnp.float32)] * 2,
            [pltpu.VMEM((2 * _EB,), jnp.int32)] * 2,
            pltpu.VMEM_SHARED((r_acc, 128), jnp.float32),
            pltpu.SemaphoreType.DMA,
            [pltpu.SemaphoreType.DMA] * 2,
            [pltpu.SemaphoreType.DMA] * 2,
            pltpu.SemaphoreType.DMA,
            [pltpu.SemaphoreType.DMA] * 2,
        ],
        compiler_params=pltpu.CompilerParams(needs_layout_passes=False),
    )
    def edge_kernel(ep_hbm, q_hbm, k_hbm, v_hbm, out_hbm,
                    ep, giq, gis, qv, kv, vv, sdv, sdi, acc,
                    isem, gqs, gks, gvs, sss):
        cid = lax.axis_index("c")
        sid = lax.axis_index("s")
        wid = sid * 2 + cid
        zeros16 = jnp.zeros((16,), jnp.float32)
        lanes = jnp.arange(16, dtype=jnp.int32)

        def zrow(r, _):
            for j in range(8):
                sdv[0][r, pl.ds(j * 16, 16)] = zeros16
                sdv[1][r, pl.ds(j * 16, 16)] = zeros16
            return 0
        lax.fori_loop(0, 2 * _EB, zrow, 0)

        row0 = pl.multiple_of(sid * rows_per, 8)
        off = 0
        while off < rows_per:
            c = min(2 * _EB, rows_per - off)
            pltpu.sync_copy(sdv[0].at[pl.ds(0, c)], acc.at[pl.ds(row0 + off, c)])
            off += c
        if tail:
            @pl.when(sid == 0)
            def _zt():
                pltpu.sync_copy(sdv[0].at[pl.ds(0, tail)],
                                acc.at[pl.ds(rows_per * 16, tail)])
        plsc.subcore_barrier()

        def idx_fetch(t):
            ebase = pl.multiple_of((wid * nblk_w + t) * _EB, 8)
            pltpu.async_copy(ep_hbm.at[pl.ds(ebase, _EB)], ep, isem)

        def slot(t, p):
            pltpu.make_async_copy(ep_hbm.at[pl.ds(0, _EB)], ep, isem).wait()
            c0 = jnp.full((16,), 0, jnp.int32)
            c1 = jnp.full((16,), 1, jnp.int32)
            for g in range(ng):
                rows = g * 16 + lanes
                gis[p][pl.ds(g * 16, 16)] = plsc.load_gather(ep, [rows, c0])
                giq[p][pl.ds(g * 16, 16)] = plsc.load_gather(ep, [rows, c1])
            pltpu.async_copy(q_hbm.at[giq[p]], qv[p], gqs[p])
            pltpu.async_copy(k_hbm.at[gis[p]], kv[p], gks[p])
            idx_fetch(t + 1)

        def wait_scatter(p):
            pltpu.make_async_copy(sdv[p], acc.at[sdi[p]], sss[p]).wait()

        def _treesum(vals):
            while len(vals) > 1:
                vals = [a + b for a, b in zip(vals[::2], vals[1::2])]
            return vals[0]

        def process(t, p, first):
            if not first:
                wait_scatter(p)
                for g in range(ng):
                    dold = sdi[p][pl.ds(g * 16, 16)]
                    dcol = (dold & 15) * 8
                    rows = _EB + g * 16 + lanes
                    for h in range(_H):
                        plsc.store_scatter(sdv[p], [rows, dcol + h], zeros16)
            pltpu.make_async_copy(q_hbm.at[giq[p]], qv[p], gqs[p]).wait()
            pltpu.make_async_copy(k_hbm.at[gis[p]], kv[p], gks[p]).wait()
            padf = jnp.where((wid * nblk_w + t) * _EB < e_real, 1.0, 0.0)
            for g in range(ng):
                rs = pl.ds(g * 16, 16)
                rows = g * 16 + lanes
                dlan = giq[p][rs]
                sdi[p][rs] = dlan
                sdi[p][pl.ds(_EB + g * 16, 16)] = n + lax.shift_right_logical(dlan, 4)
                dcol = (dlan & 15) * 8

                def _score(hh, _):
                    for u in range(2):
                        h = hh * 2 + u
                        c0 = h * _DK
                        parts = []
                        for c in range(4):
                            cs = [jnp.full((16,), 4 * c + j, jnp.int32) + c0
                                  for j in range(4)]
                            qs = [plsc.load_gather(qv[p], [rows, cj]) for cj in cs]
                            ks = [plsc.load_gather(kv[p], [rows, cj]) for cj in cs]
                            parts.append(_treesum([a * b for a, b in zip(qs, ks)]))
                        ex = jnp.exp(_treesum(parts)) * padf
                        plsc.store_scatter(sdv[p], [_EB + rows, dcol + h], ex)
                    return 0
                lax.fori_loop(0, _H // 2, _score, 0)
            pltpu.make_async_copy(v_hbm.at[gis[p]], vv, gvs).wait()
            for g in range(ng):
                rows = g * 16 + lanes
                dlan = giq[p][pl.ds(g * 16, 16)]
                dcol = (dlan & 15) * 8

                def _msg(hh, _):
                    for u in range(2):
                        h = hh * 2 + u
                        c0 = h * _DK
                        ex = plsc.load_gather(sdv[p], [_EB + rows, dcol + h])
                        for c in range(4):
                            cs = [jnp.full((16,), 4 * c + j, jnp.int32) + c0
                                  for j in range(4)]
                            vs = [plsc.load_gather(vv, [rows, cj]) for cj in cs]
                            for j in range(4):
                                plsc.store_scatter(sdv[p], [rows, cs[j]], vs[j] * ex)
                    return 0
                lax.fori_loop(0, _H // 2, _msg, 0)
            pltpu.async_copy(sdv[p], acc.at[sdi[p]], sss[p], add=True)
            pltpu.async_copy(v_hbm.at[gis[1 - p]], vv, gvs)

        idx_fetch(0)
        slot(0, 0)
        pltpu.async_copy(v_hbm.at[gis[0]], vv, gvs)
        slot(1, 1)
        process(0, 0, True)
        slot(2, 0)
        process(1, 1, True)

        def pair(i, _):
            slot(2 * i + 1, 1)
            process(2 * i, 0, False)
            slot(2 * i + 2, 0)
            process(2 * i + 1, 1, False)
            return 0
        lax.fori_loop(1, npairs, pair, 0)
        process(nblk_w - 1, 0, False)
        wait_scatter(0)
        wait_scatter(1)
        pltpu.make_async_copy(v_hbm.at[gis[1]], vv, gvs).wait()
        pltpu.make_async_copy(ep_hbm.at[pl.ds(0, _EB)], ep, isem).wait()

        plsc.subcore_barrier()
        pltpu.sync_copy(acc.at[pl.ds(row0, rows_per)],
                        out_hbm.at[cid, pl.ds(row0, rows_per)])
        if tail:
            @pl.when(sid == 0)
            def _ct():
                pltpu.sync_copy(acc.at[pl.ds(rows_per * 16, tail)],
                                out_hbm.at[cid, pl.ds(rows_per * 16, tail)])

    return edge_kernel(pairs, q_eff, k_eff, v_eff)


def kernel(x, edge_index, Wk, bk, Wq, bq, Wv, bv, Wa, ba, rel_att, rel_pri, rel_msg, skip):
    n, d = x.shape
    h, dk, _ = rel_att.shape
    ratt_bd = jax.scipy.linalg.block_diag(*[rel_att[i] for i in range(h)])
    rmsg_bd = jax.scipy.linalg.block_diag(*[rel_msg[i] for i in range(h)])
    qscale = jnp.repeat(rel_pri, dk) / math.sqrt(dk)
    q_eff, k_eff, v_eff = _qkv_pallas(x, Wq, bq, qscale, Wk, bk, ratt_bd, Wv, bv, rmsg_bd)
    e = edge_index.shape[1]
    grp_e = _EB * _NW
    nblk_w = -(-e // grp_e)
    if nblk_w % 2 == 0:
        nblk_w += 1
    e_pad = nblk_w * grp_e
    pairs = jnp.zeros((e_pad + 2 * _EB, 2), jnp.int32)
    pairs = pairs.at[:e, 0].set(edge_index[0]).at[:e, 1].set(edge_index[1])
    acc = _edge_call(pairs, q_eff, k_eff, v_eff, n, e)
    nden = (n + 15) // 16
    den = acc[:, n:n + nden, :].reshape(2, nden * 16, _H)[:, :n, :]
    return _final_pallas(acc[:, :n, :], den, x, Wa, ba, skip)


# speedup vs baseline: 1.1042x; 1.0236x over previous
import functools
import math

import jax
import jax.numpy as jnp
from jax import lax
from jax.experimental import pallas as pl
from jax.experimental.pallas import tpu as pltpu
from jax.experimental.pallas import tpu_sc as plsc

_DK = 16
_H = 8
_ACCW = 144
_EB = 32
_NW = 32


def _qkv_pallas(x, wq, bq, qscale, wk, bk, ratt_bd, wv, bv, rmsg_bd, *, interpret=False):
    n, d = x.shape
    blk = 1000
    hi = lax.Precision.HIGHEST
    dn = (((1,), (1,)), ((), ()))

    def body(x_ref, wq_ref, bq_ref, qs_ref, wk_ref, bk_ref, ra_ref, wv_ref,
             bv_ref, rm_ref, q_out, k_out, v_out):
        xb = x_ref[...]
        q = lax.dot_general(xb, wq_ref[...], dn, precision=hi)
        q_out[...] = (q + bq_ref[...]) * qs_ref[...]
        k = lax.dot_general(xb, wk_ref[...], dn, precision=hi) + bk_ref[...]
        k_out[...] = jnp.dot(k, ra_ref[...], precision=hi)
        v = lax.dot_general(xb, wv_ref[...], dn, precision=hi) + bv_ref[...]
        v_out[...] = jnp.dot(v, rm_ref[...], precision=hi)

    def full(shape):
        return pl.BlockSpec(shape, lambda i: tuple(0 for _ in shape))

    return pl.pallas_call(
        body,
        grid=(n // blk,),
        in_specs=[
            pl.BlockSpec((blk, d), lambda i: (i, 0)),
            full((d, d)), full((1, d)), full((1, d)),
            full((d, d)), full((1, d)), full((d, d)),
            full((d, d)), full((1, d)), full((d, d)),
        ],
        out_specs=[
            pl.BlockSpec((blk, d), lambda i: (i, 0)),
            pl.BlockSpec((blk, d), lambda i: (i, 0)),
            pl.BlockSpec((blk, d), lambda i: (i, 0)),
        ],
        out_shape=[
            jax.ShapeDtypeStruct((n, d), jnp.float32),
            jax.ShapeDtypeStruct((n, d), jnp.float32),
            jax.ShapeDtypeStruct((n, d), jnp.float32),
        ],
        interpret=interpret,
    )(x, wq, bq.reshape(1, d), qscale.reshape(1, d), wk, bk.reshape(1, d),
      ratt_bd, wv, bv.reshape(1, d), rmsg_bd)


def _final_pallas(num, den, x, wa, ba, skip, *, interpret=False):
    n, d = x.shape
    blk = 1000
    hi = lax.Precision.HIGHEST
    dn = (((1,), (1,)), ((), ()))

    def body(num_ref, den_ref, x_ref, wa_ref, ba_ref, skip_ref, out_ref):
        nm = num_ref[0] + num_ref[1]
        den8 = den_ref[0] + den_ref[1]
        hh = lax.broadcasted_iota(jnp.int32, (_H, d), 0)
        cc = lax.broadcasted_iota(jnp.int32, (_H, d), 1)
        sel = jnp.where((cc // _DK) == hh, 1.0, 0.0)
        den_rep = jnp.dot(den8, sel, precision=hi)
        den_rep = jnp.where(den_rep > 0.0, den_rep, 1.0)
        t = nm / den_rep
        out = lax.dot_general(t, wa_ref[...], dn, precision=hi) + ba_ref[...]
        alpha = 1.0 / (1.0 + jnp.exp(-skip_ref[...]))
        out_ref[...] = out * alpha + x_ref[...] * (1.0 - alpha)

    return pl.pallas_call(
        body,
        grid=(n // blk,),
        in_specs=[
            pl.BlockSpec((2, blk, d), lambda i: (0, i, 0)),
            pl.BlockSpec((2, blk, _H), lambda i: (0, i, 0)),
            pl.BlockSpec((blk, d), lambda i: (i, 0)),
            pl.BlockSpec((d, d), lambda i: (0, 0)),
            pl.BlockSpec((1, d), lambda i: (0, 0)),
            pl.BlockSpec((1, 1), lambda i: (0, 0)),
        ],
        out_specs=pl.BlockSpec((blk, d), lambda i: (i, 0)),
        out_shape=jax.ShapeDtypeStruct((n, d), jnp.float32),
        interpret=interpret,
    )(num, den, x, wa, ba.reshape(1, d), skip.reshape(1, 1))


def _edge_call(pairs, q_eff, k_eff, v_eff, n, e_real):
    e_pad = pairs.shape[0] - 2 * _EB
    d = q_eff.shape[1]
    nblk_w = e_pad // (_EB * _NW)
    assert nblk_w * _EB * _NW == e_pad and nblk_w % 2 == 1 and nblk_w >= 3
    assert e_real % _EB == 0
    npairs = (nblk_w - 1) // 2
    nden = (n + 15) // 16
    r_acc = (n + nden + 7) // 8 * 8
    rows_per = (r_acc // 16) // 8 * 8
    tail = r_acc - rows_per * 16
    assert rows_per % 8 == 0 and tail % 8 == 0
    mesh = plsc.VectorSubcoreMesh(core_axis_name="c", subcore_axis_name="s")

    ng = _EB // 16

    @functools.partial(
        pl.kernel,
        out_type=jax.ShapeDtypeStruct((2, r_acc, 128), jnp.float32),
        mesh=mesh,
        scratch_types=[
            pltpu.VMEM((_EB, 2), jnp.int32),
            [pltpu.VMEM((_EB,), jnp.int32)] * 2,
            [pltpu.VMEM((_EB,), jnp.int32)] * 2,
            [pltpu.VMEM((_EB, 128), jnp.float32)] * 2,
            [pltpu.VMEM((_EB, 128), jnp.float32)] * 2,
            pltpu.VMEM((_EB, 128), jnp.float32),
            [pltpu.VMEM((2 * _EB, 128), jnp.float32)] * 2,
            [pltpu.VMEM((2 * _EB,), jnp.int32)] * 2,
            pltpu.VMEM_SHARED((r_acc, 128), jnp.float32),
            pltpu.SemaphoreType.DMA,
            [pltpu.SemaphoreType.DMA] * 2,
            [pltpu.SemaphoreType.DMA] * 2,
            pltpu.SemaphoreType.DMA,
            [pltpu.SemaphoreType.DMA] * 2,
        ],
        compiler_params=pltpu.CompilerParams(needs_layout_passes=False),
    )
    def edge_kernel(ep_hbm, q_hbm, k_hbm, v_hbm, out_hbm,
                    ep, giq, gis, qv, kv, vv, sdv, sdi, acc,
                    isem, gqs, gks, gvs, sss):
        cid = lax.axis_index("c")
        sid = lax.axis_index("s")
        wid = sid * 2 + cid
        zeros16 = jnp.zeros((16,), jnp.float32)
        lanes = jnp.arange(16, dtype=jnp.int32)

        def zrow(r, _):
            for j in range(8):
                sdv[0][r, pl.ds(j * 16, 16)] = zeros16
                sdv[1][r, pl.ds(j * 16, 16)] = zeros16
            return 0
        lax.fori_loop(0, 2 * _EB, zrow, 0)

        row0 = pl.multiple_of(sid * rows_per, 8)
        off = 0
        while off < rows_per:
            c = min(2 * _EB, rows_per - off)
            pltpu.sync_copy(sdv[0].at[pl.ds(0, c)], acc.at[pl.ds(row0 + off, c)])
            off += c
        if tail:
            @pl.when(sid == 0)
            def _zt():
                pltpu.sync_copy(sdv[0].at[pl.ds(0, tail)],
                                acc.at[pl.ds(rows_per * 16, tail)])
        plsc.subcore_barrier()

        def idx_fetch(t):
            ebase = pl.multiple_of((wid * nblk_w + t) * _EB, 8)
            pltpu.async_copy(ep_hbm.at[pl.ds(ebase, _EB)], ep, isem)

        def slot(t, p):
            pltpu.make_async_copy(ep_hbm.at[pl.ds(0, _EB)], ep, isem).wait()
            c0 = jnp.full((16,), 0, jnp.int32)
            c1 = jnp.full((16,), 1, jnp.int32)
            for g in range(ng):
                rows = g * 16 + lanes
                gis[p][pl.ds(g * 16, 16)] = plsc.load_gather(ep, [rows, c0])
                giq[p][pl.ds(g * 16, 16)] = plsc.load_gather(ep, [rows, c1])
            pltpu.async_copy(q_hbm.at[giq[p]], qv[p], gqs[p])
            pltpu.async_copy(k_hbm.at[gis[p]], kv[p], gks[p])
            idx_fetch(t + 1)

        def wait_scatter(p):
            pltpu.make_async_copy(sdv[p], acc.at[sdi[p]], sss[p]).wait()

        def _treesum(vals):
            while len(vals) > 1:
                vals = [a + b for a, b in zip(vals[::2], vals[1::2])]
            return vals[0]

        def process(t, p, first):
            if not first:
                wait_scatter(p)
                for g in range(ng):
                    dold = sdi[p][pl.ds(g * 16, 16)]
                    dcol = (dold & 15) * 8
                    rows = _EB + g * 16 + lanes
                    for h in range(_H):
                        plsc.store_scatter(sdv[p], [rows, dcol + h], zeros16)
            pltpu.make_async_copy(q_hbm.at[giq[p]], qv[p], gqs[p]).wait()
            pltpu.make_async_copy(k_hbm.at[gis[p]], kv[p], gks[p]).wait()
            padf = jnp.where((wid * nblk_w + t) * _EB < e_real, 1.0, 0.0)
            for g in range(ng):
                rs = pl.ds(g * 16, 16)
                rows = g * 16 + lanes
                dlan = giq[p][rs]
                sdi[p][rs] = dlan
                sdi[p][pl.ds(_EB + g * 16, 16)] = n + lax.shift_right_logical(dlan, 4)
                dcol = (dlan & 15) * 8

                def _score(hh, _):
                    for u in range(4):
                        h = hh * 4 + u
                        c0 = h * _DK
                        parts = []
                        for c in range(4):
                            cs = [jnp.full((16,), 4 * c + j, jnp.int32) + c0
                                  for j in range(4)]
                            qs = [plsc.load_gather(qv[p], [rows, cj]) for cj in cs]
                            ks = [plsc.load_gather(kv[p], [rows, cj]) for cj in cs]
                            parts.append(_treesum([a * b for a, b in zip(qs, ks)]))
                        ex = jnp.exp(_treesum(parts)) * padf
                        plsc.store_scatter(sdv[p], [_EB + rows, dcol + h], ex)
                    return 0
                lax.fori_loop(0, _H // 4, _score, 0)
            pltpu.make_async_copy(v_hbm.at[gis[p]], vv, gvs).wait()
            for g in range(ng):
                rows = g * 16 + lanes
                dlan = giq[p][pl.ds(g * 16, 16)]
                dcol = (dlan & 15) * 8

                def _msg(hh, _):
                    for u in range(4):
                        h = hh * 4 + u
                        c0 = h * _DK
                        ex = plsc.load_gather(sdv[p], [_EB + rows, dcol + h])
                        for c in range(4):
                            cs = [jnp.full((16,), 4 * c + j, jnp.int32) + c0
                                  for j in range(4)]
                            vs = [plsc.load_gather(vv, [rows, cj]) for cj in cs]
                            for j in range(4):
                                plsc.store_scatter(sdv[p], [rows, cs[j]], vs[j] * ex)
                    return 0
                lax.fori_loop(0, _H // 4, _msg, 0)
            pltpu.async_copy(sdv[p], acc.at[sdi[p]], sss[p], add=True)
            pltpu.async_copy(v_hbm.at[gis[1 - p]], vv, gvs)

        idx_fetch(0)
        slot(0, 0)
        pltpu.async_copy(v_hbm.at[gis[0]], vv, gvs)
        slot(1, 1)
        process(0, 0, True)
        slot(2, 0)
        process(1, 1, True)

        def pair(i, _):
            slot(2 * i + 1, 1)
            process(2 * i, 0, False)
            slot(2 * i + 2, 0)
            process(2 * i + 1, 1, False)
            return 0
        lax.fori_loop(1, npairs, pair, 0)
        process(nblk_w - 1, 0, False)
        wait_scatter(0)
        wait_scatter(1)
        pltpu.make_async_copy(v_hbm.at[gis[1]], vv, gvs).wait()
        pltpu.make_async_copy(ep_hbm.at[pl.ds(0, _EB)], ep, isem).wait()

        plsc.subcore_barrier()
        pltpu.sync_copy(acc.at[pl.ds(row0, rows_per)],
                        out_hbm.at[cid, pl.ds(row0, rows_per)])
        if tail:
            @pl.when(sid == 0)
            def _ct():
                pltpu.sync_copy(acc.at[pl.ds(rows_per * 16, tail)],
                                out_hbm.at[cid, pl.ds(rows_per * 16, tail)])

    return edge_kernel(pairs, q_eff, k_eff, v_eff)


def kernel(x, edge_index, Wk, bk, Wq, bq, Wv, bv, Wa, ba, rel_att, rel_pri, rel_msg, skip):
    n, d = x.shape
    h, dk, _ = rel_att.shape
    ratt_bd = jax.scipy.linalg.block_diag(*[rel_att[i] for i in range(h)])
    rmsg_bd = jax.scipy.linalg.block_diag(*[rel_msg[i] for i in range(h)])
    qscale = jnp.repeat(rel_pri, dk) / math.sqrt(dk)
    q_eff, k_eff, v_eff = _qkv_pallas(x, Wq, bq, qscale, Wk, bk, ratt_bd, Wv, bv, rmsg_bd)
    e = edge_index.shape[1]
    grp_e = _EB * _NW
    nblk_w = -(-e // grp_e)
    if nblk_w % 2 == 0:
        nblk_w += 1
    e_pad = nblk_w * grp_e
    pairs = jnp.zeros((e_pad + 2 * _EB, 2), jnp.int32)
    pairs = pairs.at[:e, 0].set(edge_index[0]).at[:e, 1].set(edge_index[1])
    acc = _edge_call(pairs, q_eff, k_eff, v_eff, n, e)
    nden = (n + 15) // 16
    den = acc[:, n:n + nden, :].reshape(2, nden * 16, _H)[:, :n, :]
    return _final_pallas(acc[:, :n, :], den, x, Wa, ba, skip)
